# Initial kernel scaffold; baseline (speedup 1.0000x reference)
import jax
import jax.numpy as jnp
from jax.experimental import pallas as pl


def kernel(x, W, b):
    raise NotImplementedError("write your pallas kernel here")


# speedup vs baseline: 2.0993x; 2.0993x over previous
import functools

import jax
import jax.numpy as jnp
from jax.experimental import pallas as pl
from jax.experimental.pallas import tpu as pltpu

D_MODEL = 768
NUM_EXPERTS = 64
N_TOKENS = 32768
BT = 1024


def _router_kernel(x_ref, w_ref, b_ref, probs_ref, idx_ref):
    logits = jax.lax.dot_general(
        x_ref[:], w_ref[:],
        dimension_numbers=(((1,), (1,)), ((), ())),
        preferred_element_type=jnp.float32,
    ) + b_ref[:]
    lane = jax.lax.broadcasted_iota(jnp.int32, logits.shape, 1)

    v0 = jnp.max(logits, axis=1, keepdims=True)
    i0 = jnp.min(jnp.where(logits == v0, lane, NUM_EXPERTS), axis=1,
                 keepdims=True)
    masked = jnp.where(lane == i0, -jnp.inf, logits)
    v1 = jnp.max(masked, axis=1, keepdims=True)
    i1 = jnp.min(jnp.where(masked == v1, lane, NUM_EXPERTS), axis=1,
                 keepdims=True)

    e = jnp.exp(v1 - v0)
    p0 = 1.0 / (1.0 + e)
    p1 = e * p0

    probs_ref[:] = jnp.concatenate([p0, p1], axis=1)
    idx_ref[:] = jnp.concatenate([i0, i1], axis=1)


@functools.partial(jax.jit, static_argnames=())
def kernel(x, W, b):
    n = x.shape[0]
    grid = (n // BT,)
    probs, idx = pl.pallas_call(
        _router_kernel,
        grid=grid,
        in_specs=[
            pl.BlockSpec((BT, D_MODEL), lambda i: (i, 0)),
            pl.BlockSpec((NUM_EXPERTS, D_MODEL), lambda i: (0, 0)),
            pl.BlockSpec((1, NUM_EXPERTS), lambda i: (0, 0)),
        ],
        out_specs=[
            pl.BlockSpec((BT, 2), lambda i: (i, 0)),
            pl.BlockSpec((BT, 2), lambda i: (i, 0)),
        ],
        out_shape=[
            jax.ShapeDtypeStruct((n, 2), jnp.float32),
            jax.ShapeDtypeStruct((n, 2), jnp.int32),
        ],
        compiler_params=pltpu.CompilerParams(
            dimension_semantics=("arbitrary",),
        ),
    )(x, W, b.reshape(1, NUM_EXPERTS))
    return (probs, idx)


# speedup vs baseline: 2.4079x; 1.1470x over previous
import functools

import jax
import jax.numpy as jnp
from jax.experimental import pallas as pl
from jax.experimental.pallas import tpu as pltpu

D_MODEL = 768
NUM_EXPERTS = 64
N_TOKENS = 32768
BT = 2048


def _router_kernel(x_ref, w_ref, b_ref, probs_ref, idx_ref):
    logits = jax.lax.dot_general(
        x_ref[:], w_ref[:],
        dimension_numbers=(((1,), (1,)), ((), ())),
        preferred_element_type=jnp.float32,
    ) + b_ref[:]
    lane = jax.lax.broadcasted_iota(jnp.int32, logits.shape, 1)

    v0 = jnp.max(logits, axis=1, keepdims=True)
    i0 = jnp.min(jnp.where(logits == v0, lane, NUM_EXPERTS), axis=1,
                 keepdims=True)
    masked = jnp.where(lane == i0, -jnp.inf, logits)
    v1 = jnp.max(masked, axis=1, keepdims=True)
    i1 = jnp.min(jnp.where(masked == v1, lane, NUM_EXPERTS), axis=1,
                 keepdims=True)

    e = jnp.exp(v1 - v0)
    p0 = 1.0 / (1.0 + e)
    p1 = e * p0

    probs_ref[:] = jnp.concatenate([p0, p1], axis=1)
    idx_ref[:] = jnp.concatenate([i0, i1], axis=1)


@functools.partial(jax.jit, static_argnames=())
def kernel(x, W, b):
    n = x.shape[0]
    grid = (n // BT,)
    probs, idx = pl.pallas_call(
        _router_kernel,
        grid=grid,
        in_specs=[
            pl.BlockSpec((BT, D_MODEL), lambda i: (i, 0)),
            pl.BlockSpec((NUM_EXPERTS, D_MODEL), lambda i: (0, 0)),
            pl.BlockSpec((1, NUM_EXPERTS), lambda i: (0, 0)),
        ],
        out_specs=[
            pl.BlockSpec((BT, 2), lambda i: (i, 0)),
            pl.BlockSpec((BT, 2), lambda i: (i, 0)),
        ],
        out_shape=[
            jax.ShapeDtypeStruct((n, 2), jnp.float32),
            jax.ShapeDtypeStruct((n, 2), jnp.int32),
        ],
        compiler_params=pltpu.CompilerParams(
            dimension_semantics=("arbitrary",),
        ),
    )(x, W, b.reshape(1, NUM_EXPERTS))
    return (probs, idx)


# speedup vs baseline: 2.5919x; 1.0764x over previous
import functools

import jax
import jax.numpy as jnp
from jax.experimental import pallas as pl
from jax.experimental.pallas import tpu as pltpu

D_MODEL = 768
NUM_EXPERTS = 64
N_TOKENS = 32768
BT = 4096


def _router_kernel(x_ref, w_ref, b_ref, probs_ref, idx_ref):
    logits = jax.lax.dot_general(
        x_ref[:], w_ref[:],
        dimension_numbers=(((1,), (1,)), ((), ())),
        preferred_element_type=jnp.float32,
    ) + b_ref[:]
    lane = jax.lax.broadcasted_iota(jnp.int32, logits.shape, 1)

    v0 = jnp.max(logits, axis=1, keepdims=True)
    i0 = jnp.min(jnp.where(logits == v0, lane, NUM_EXPERTS), axis=1,
                 keepdims=True)
    masked = jnp.where(lane == i0, -jnp.inf, logits)
    v1 = jnp.max(masked, axis=1, keepdims=True)
    i1 = jnp.min(jnp.where(masked == v1, lane, NUM_EXPERTS), axis=1,
                 keepdims=True)

    e = jnp.exp(v1 - v0)
    p0 = 1.0 / (1.0 + e)
    p1 = e * p0

    probs_ref[:] = jnp.concatenate([p0, p1], axis=1)
    idx_ref[:] = jnp.concatenate([i0, i1], axis=1)


@functools.partial(jax.jit, static_argnames=())
def kernel(x, W, b):
    n = x.shape[0]
    grid = (n // BT,)
    probs, idx = pl.pallas_call(
        _router_kernel,
        grid=grid,
        in_specs=[
            pl.BlockSpec((BT, D_MODEL), lambda i: (i, 0)),
            pl.BlockSpec((NUM_EXPERTS, D_MODEL), lambda i: (0, 0)),
            pl.BlockSpec((1, NUM_EXPERTS), lambda i: (0, 0)),
        ],
        out_specs=[
            pl.BlockSpec((BT, 2), lambda i: (i, 0)),
            pl.BlockSpec((BT, 2), lambda i: (i, 0)),
        ],
        out_shape=[
            jax.ShapeDtypeStruct((n, 2), jnp.float32),
            jax.ShapeDtypeStruct((n, 2), jnp.int32),
        ],
        compiler_params=pltpu.CompilerParams(
            dimension_semantics=("arbitrary",),
        ),
    )(x, W, b.reshape(1, NUM_EXPERTS))
    return (probs, idx)
